# baseline (speedup 1.0000x reference)
import functools

import jax
import jax.numpy as jnp
from jax import lax
from jax.experimental import pallas as pl
from jax.experimental.pallas import tpu as pltpu
from jax.experimental.pallas import tpu_sc as plsc

N = 2048
H = 1024
E = 8
F = 4096
CAP = 320
FB = 512
NFB = F // FB

try:
  _SC_INFO = plsc.get_sparse_core_info()
  _NC, _NS = _SC_INFO.num_cores, _SC_INFO.num_subcores
except Exception:
  _NC, _NS = 2, 16
_NW = _NC * _NS
_BPW = (E * CAP) // _NW


def _excl_cumsum_tokens(m):
  tri = (jax.lax.broadcasted_iota(jnp.int32, (128, 128), 1)
         < jax.lax.broadcasted_iota(jnp.int32, (128, 128), 0)).astype(jnp.float32)
  parts = []
  off = jnp.zeros((1, E), jnp.float32)
  for c in range(N // 128):
    blk = m[c * 128:(c + 1) * 128, :]
    within = jax.lax.dot_general(tri, blk, (((1,), (0,)), ((), ())),
                                 preferred_element_type=jnp.float32)
    parts.append(within + off)
    off = off + jnp.sum(blk, axis=0, keepdims=True)
  return jnp.concatenate(parts, axis=0)


def _route_kernel(x_ref, wg_ref, idx_ref, cw_ref, rselt_ref, logits_ref):
  e = pl.program_id(0)

  @pl.when(e == 0)
  def _init():
    logits = jax.lax.dot_general(x_ref[...], wg_ref[...],
                                 (((1,), (1,)), ((), ())),
                                 preferred_element_type=jnp.float32)
    logits_ref[...] = logits
    b = jax.lax.bitcast_convert_type(logits, jnp.int32)
    skey = b ^ (jax.lax.shift_right_arithmetic(b, 31) & jnp.int32(0x7FFFFFFF))
    sprefix = jnp.full((1, E), -2**31, jnp.int32)
    for bit in range(31, -1, -1):
      bitc = jnp.int32(-2**31) if bit == 31 else jnp.int32(1 << bit)
      scand = sprefix ^ bitc
      cnt = jnp.sum((skey >= scand).astype(jnp.int32), axis=0, keepdims=True)
      sprefix = jnp.where(cnt >= CAP, scand, sprefix)
    gt = skey > sprefix
    tie = skey == sprefix
    n_gt = jnp.sum(gt.astype(jnp.int32), axis=0, keepdims=True)
    need = (CAP - n_gt).astype(jnp.float32)
    tie_rank = _excl_cumsum_tokens(tie.astype(jnp.float32))
    sel = gt | (tie & (tie_rank < need))
    rank = _excl_cumsum_tokens(sel.astype(jnp.float32))
    rselt_ref[...] = jnp.where(sel, rank, -1.0)

  logits = logits_ref[...]
  m = jnp.max(logits, axis=1, keepdims=True)
  ex = jnp.exp(logits - m)
  probs = ex / jnp.sum(ex, axis=1, keepdims=True)
  lane_e = jax.lax.broadcasted_iota(jnp.int32, (N, E), 1) == e
  pe_col = jnp.sum(jnp.where(lane_e, probs, 0.0), axis=1, keepdims=True)
  rsel_col = jnp.max(jnp.where(lane_e, rselt_ref[...], -2.0), axis=1,
                     keepdims=True).astype(jnp.int32)
  pt = (rsel_col == jax.lax.broadcasted_iota(jnp.int32, (N, CAP), 1)
        ).astype(jnp.float32)
  tok = jax.lax.broadcasted_iota(jnp.int32, (N, 1), 0).astype(jnp.float32)
  both = jnp.concatenate([tok, pe_col], axis=1)
  res = jax.lax.dot_general(pt, both, (((0,), (0,)), ((), ())),
                            preferred_element_type=jnp.float32)
  idx_ref[0] = res[:, 0:1]
  cw_ref[0] = res[:, 1:2]


def _sc_gather_kernel(x_hbm, idx_hbm, out_hbm, idx_v, rows_v, sem):
  wid = lax.axis_index("s") * _NC + lax.axis_index("c")
  base = wid * _BPW
  pltpu.sync_copy(idx_hbm.at[pl.ds(base, _BPW)], idx_v)
  pltpu.async_copy(x_hbm.at[idx_v], rows_v, sem).wait()
  pltpu.sync_copy(rows_v, out_hbm.at[pl.ds(base, _BPW)])


_EPG = 4


def _ffn_scatter_kernel(g_ref, cw_ref, w1_ref, w2_ref, rselt_ref, out_ref,
                        gs_ref, acc_ref, eoutg_ref):
  e = pl.program_id(0)
  f = pl.program_id(1)

  @pl.when(jnp.logical_and(e == 0, f == 0))
  def _():
    out_ref[...] = jnp.zeros_like(out_ref)

  @pl.when(f == 0)
  def _scale():
    gs_ref[...] = g_ref[0] * cw_ref[0]

  hmid = jnp.maximum(
      jax.lax.dot_general(gs_ref[...], w1_ref[0], (((1,), (0,)), ((), ())),
                          preferred_element_type=jnp.float32), 0.0)
  contrib = jax.lax.dot_general(hmid, w2_ref[0], (((1,), (0,)), ((), ())),
                                preferred_element_type=jnp.float32)

  @pl.when(f == 0)
  def _():
    acc_ref[...] = contrib

  @pl.when(f > 0)
  def _():
    acc_ref[...] = acc_ref[...] + contrib

  @pl.when(f == NFB - 1)
  def _stash():
    eoutg_ref[pl.ds((e % _EPG) * CAP, CAP), :] = acc_ref[...]

  @pl.when(jnp.logical_and(e % _EPG == _EPG - 1, f == NFB - 1))
  def _scatter():
    slot_iota = jax.lax.broadcasted_iota(jnp.int32, (N, CAP), 1)
    slabs = []
    for j in range(_EPG):
      ej = e - (_EPG - 1) + j
      lane_e = jax.lax.broadcasted_iota(jnp.int32, (N, E), 1) == ej
      rsel_col = jnp.max(jnp.where(lane_e, rselt_ref[...], -2.0), axis=1,
                         keepdims=True).astype(jnp.int32)
      slabs.append((rsel_col == slot_iota).astype(jnp.float32))
    pg = jnp.concatenate(slabs, axis=1)
    out_ref[...] += jax.lax.dot_general(pg, eoutg_ref[...],
                                        (((1,), (0,)), ((), ())),
                                        preferred_element_type=jnp.float32)


def kernel(x, Wg, W1, W2):
  idx_f, cw, rselt = pl.pallas_call(
      _route_kernel,
      grid=(E,),
      in_specs=[
          pl.BlockSpec((N, H), lambda e: (0, 0)),
          pl.BlockSpec((E, H), lambda e: (0, 0)),
      ],
      out_specs=[
          pl.BlockSpec((1, CAP, 1), lambda e: (e, 0, 0)),
          pl.BlockSpec((1, CAP, 1), lambda e: (e, 0, 0)),
          pl.BlockSpec((N, E), lambda e: (0, 0)),
      ],
      out_shape=[
          jax.ShapeDtypeStruct((E, CAP, 1), jnp.float32),
          jax.ShapeDtypeStruct((E, CAP, 1), jnp.float32),
          jax.ShapeDtypeStruct((N, E), jnp.float32),
      ],
      scratch_shapes=[pltpu.VMEM((N, E), jnp.float32)],
  )(x, Wg)

  idx = idx_f.reshape(E * CAP).astype(jnp.int32)

  mesh = plsc.VectorSubcoreMesh(core_axis_name="c", subcore_axis_name="s")
  gathered = pl.kernel(
      _sc_gather_kernel,
      mesh=mesh,
      out_type=jax.ShapeDtypeStruct((E * CAP, H), jnp.float32),
      scratch_types=[
          pltpu.VMEM((_BPW,), jnp.int32),
          pltpu.VMEM((_BPW, H), jnp.float32),
          pltpu.SemaphoreType.DMA,
      ],
  )(x, idx)

  out = pl.pallas_call(
      _ffn_scatter_kernel,
      grid=(E, NFB),
      in_specs=[
          pl.BlockSpec((1, CAP, H), lambda e, f: (e, 0, 0)),
          pl.BlockSpec((1, CAP, 1), lambda e, f: (e, 0, 0)),
          pl.BlockSpec((1, H, FB), lambda e, f: (e, 0, f)),
          pl.BlockSpec((1, FB, H), lambda e, f: (e, f, 0)),
          pl.BlockSpec((N, E), lambda e, f: (0, 0)),
      ],
      out_specs=pl.BlockSpec((N, H), lambda e, f: (0, 0)),
      out_shape=jax.ShapeDtypeStruct((N, H), jnp.float32),
      scratch_shapes=[
          pltpu.VMEM((CAP, H), jnp.float32),
          pltpu.VMEM((CAP, H), jnp.float32),
          pltpu.VMEM((_EPG * CAP, H), jnp.float32),
      ],
  )(gathered.reshape(E, CAP, H), cw, W1, W2, rselt)

  expert_load = jnp.full((E,), float(CAP), jnp.float32)
  lbl = (expert_load * jnp.log(expert_load / expert_load.mean() + 1e-08)).mean()
  return out, lbl


# speedup vs baseline: 15.8113x; 15.8113x over previous
import jax
import jax.numpy as jnp
from jax.experimental import pallas as pl
from jax.experimental.pallas import tpu as pltpu

N = 2048
H = 1024
E = 8
F = 4096
CAP = 320
FB = 512
NFB = F // FB
EPG = 4


def _excl_cumsum_tokens(m):
  tri = (jax.lax.broadcasted_iota(jnp.int32, (128, 128), 1)
         < jax.lax.broadcasted_iota(jnp.int32, (128, 128), 0)).astype(jnp.float32)
  parts = []
  off = jnp.zeros((1, E), jnp.float32)
  for c in range(N // 128):
    blk = m[c * 128:(c + 1) * 128, :]
    within = jax.lax.dot_general(tri, blk, (((1,), (0,)), ((), ())),
                                 preferred_element_type=jnp.float32)
    parts.append(within + off)
    off = off + jnp.sum(blk, axis=0, keepdims=True)
  return jnp.concatenate(parts, axis=0)


def _rsel_col(rselt, e):
  lane_e = jax.lax.broadcasted_iota(jnp.int32, (N, E), 1) == e
  return jnp.max(jnp.where(lane_e, rselt, -2.0), axis=1,
                 keepdims=True).astype(jnp.int32)


def _moe_kernel(x_ref, wg_ref, w1_ref, w2_ref, out_ref,
                logits_ref, rselt_ref, gs_ref, acc_ref, eoutg_ref):
  e = pl.program_id(0)
  f = pl.program_id(1)

  @pl.when(jnp.logical_and(e == 0, f == 0))
  def _route():
    logits = jax.lax.dot_general(x_ref[...], wg_ref[...],
                                 (((1,), (1,)), ((), ())),
                                 preferred_element_type=jnp.float32)
    logits_ref[...] = logits
    b = jax.lax.bitcast_convert_type(logits, jnp.int32)
    skey = b ^ (jax.lax.shift_right_arithmetic(b, 31) & jnp.int32(0x7FFFFFFF))
    sprefix = jnp.full((1, E), -2**31, jnp.int32)
    for bit in range(31, -1, -1):
      bitc = jnp.int32(-2**31) if bit == 31 else jnp.int32(1 << bit)
      scand = sprefix ^ bitc
      cnt = jnp.sum((skey >= scand).astype(jnp.int32), axis=0, keepdims=True)
      sprefix = jnp.where(cnt >= CAP, scand, sprefix)
    gt = skey > sprefix
    tie = skey == sprefix
    n_gt = jnp.sum(gt.astype(jnp.int32), axis=0, keepdims=True)
    need = (CAP - n_gt).astype(jnp.float32)
    tie_rank = _excl_cumsum_tokens(tie.astype(jnp.float32))
    sel = gt | (tie & (tie_rank < need))
    rank = _excl_cumsum_tokens(sel.astype(jnp.float32))
    rselt_ref[...] = jnp.where(sel, rank, -1.0)
    out_ref[...] = jnp.zeros_like(out_ref)

  @pl.when(f == 0)
  def _gather():
    logits = logits_ref[...]
    m = jnp.max(logits, axis=1, keepdims=True)
    ex = jnp.exp(logits - m)
    probs = ex / jnp.sum(ex, axis=1, keepdims=True)
    lane_e = jax.lax.broadcasted_iota(jnp.int32, (N, E), 1) == e
    pe_col = jnp.sum(jnp.where(lane_e, probs, 0.0), axis=1, keepdims=True)
    slot_iota = jax.lax.broadcasted_iota(jnp.int32, (N, CAP), 1)
    pwt = jnp.where(_rsel_col(rselt_ref[...], e) == slot_iota,
                    pe_col, 0.0)
    gs_ref[...] = jax.lax.dot_general(pwt, x_ref[...], (((0,), (0,)), ((), ())),
                                      preferred_element_type=jnp.float32)

  hmid = jnp.maximum(
      jax.lax.dot_general(gs_ref[...], w1_ref[0], (((1,), (0,)), ((), ())),
                          preferred_element_type=jnp.float32), 0.0)
  contrib = jax.lax.dot_general(hmid, w2_ref[0], (((1,), (0,)), ((), ())),
                                preferred_element_type=jnp.float32)

  @pl.when(f == 0)
  def _():
    acc_ref[...] = contrib

  @pl.when(f > 0)
  def _():
    acc_ref[...] = acc_ref[...] + contrib

  @pl.when(f == NFB - 1)
  def _stash():
    eoutg_ref[pl.ds((e % EPG) * CAP, CAP), :] = acc_ref[...]

  @pl.when(jnp.logical_and(e % EPG == EPG - 1, f == NFB - 1))
  def _scatter():
    slot_iota = jax.lax.broadcasted_iota(jnp.int32, (N, CAP), 1)
    slabs = []
    for j in range(EPG):
      ej = e - (EPG - 1) + j
      slabs.append((_rsel_col(rselt_ref[...], ej) == slot_iota
                    ).astype(jnp.float32))
    pg = jnp.concatenate(slabs, axis=1)
    out_ref[...] += jax.lax.dot_general(pg, eoutg_ref[...],
                                        (((1,), (0,)), ((), ())),
                                        preferred_element_type=jnp.float32)


def kernel(x, Wg, W1, W2):
  out = pl.pallas_call(
      _moe_kernel,
      grid=(E, NFB),
      in_specs=[
          pl.BlockSpec((N, H), lambda e, f: (0, 0)),
          pl.BlockSpec((E, H), lambda e, f: (0, 0)),
          pl.BlockSpec((1, H, FB), lambda e, f: (e, 0, f)),
          pl.BlockSpec((1, FB, H), lambda e, f: (e, f, 0)),
      ],
      out_specs=pl.BlockSpec((N, H), lambda e, f: (0, 0)),
      out_shape=jax.ShapeDtypeStruct((N, H), jnp.float32),
      scratch_shapes=[
          pltpu.VMEM((N, E), jnp.float32),
          pltpu.VMEM((N, E), jnp.float32),
          pltpu.VMEM((CAP, H), jnp.float32),
          pltpu.VMEM((CAP, H), jnp.float32),
          pltpu.VMEM((EPG * CAP, H), jnp.float32),
      ],
  )(x, Wg, W1, W2)

  expert_load = jnp.full((E,), float(CAP), jnp.float32)
  lbl = (expert_load * jnp.log(expert_load / expert_load.mean() + 1e-08)).mean()
  return out, lbl
